# Initial kernel scaffold; baseline (speedup 1.0000x reference)
import jax
import jax.numpy as jnp
from jax.experimental import pallas as pl


def kernel(features, edge_index_e0, edge_index_e1, edge_index_e2, W_e0, b_e0, W_e1, b_e1, W_e2, b_e2):
    raise NotImplementedError("write your pallas kernel here")


# speedup vs baseline: 2.9800x; 2.9800x over previous
import jax
import jax.numpy as jnp
from jax import lax
from jax.experimental import pallas as pl
from jax.experimental.pallas import tpu as pltpu
from jax.experimental.pallas import tpu_sc as plsc

N = 10000
D = 128
E = 320000
NC = 2
NS = 16
NW = NC * NS
LANES = 128
EPAD = 327680
ROWS_ALL = EPAD // LANES
ROWS_W = ROWS_ALL // NW
SUPER = 8
NSUPER = ROWS_W // SUPER
NPAD = N + 208
NACC = N + 16
ROWS_T = 624
REM = N - NS * ROWS_T
ZROWS = 208


def _sc_body(feat_hbm, src_hbm, dst_hbm, zc_hbm,
             sums_hbm, cnts_hbm,
             acc_sh, rows_v, sidx_v, didx_v, cnt_v, sem):
  c = lax.axis_index("c")
  s = lax.axis_index("s")
  wid = c * NS + s
  ones16 = jnp.full((16,), 1.0, jnp.float32)

  for e in range(3):
    for k in range(3):
      pltpu.sync_copy(feat_hbm.at[pl.ds(N, ZROWS)],
                      acc_sh.at[pl.ds(s * ROWS_T + k * ZROWS, ZROWS)])

    @pl.when(s == NS - 1)
    def _():
      pltpu.sync_copy(feat_hbm.at[pl.ds(N, REM)],
                      acc_sh.at[pl.ds(NS * ROWS_T, REM)])

    pltpu.sync_copy(zc_hbm, cnt_v)
    plsc.subcore_barrier()

    def superchunk(g, _):
      rowbase = wid * ROWS_W + g * SUPER
      pltpu.sync_copy(src_hbm.at[e, pl.ds(rowbase, SUPER)], sidx_v)
      pltpu.sync_copy(dst_hbm.at[e, pl.ds(rowbase, SUPER)], didx_v)
      for j in range(SUPER):
        pltpu.async_copy(feat_hbm.at[sidx_v.at[j]], rows_v, sem).wait()
        pltpu.sync_copy(rows_v, acc_sh.at[didx_v.at[j]], add=True)
        for k in range(LANES // 16):
          idx = didx_v[j, pl.ds(k * 16, 16)]
          plsc.addupdate_scatter(cnt_v, [idx], ones16)
      return 0

    lax.fori_loop(0, NSUPER, superchunk, 0)
    plsc.subcore_barrier()

    pltpu.sync_copy(acc_sh.at[pl.ds(s * ROWS_T, ROWS_T)],
                    sums_hbm.at[e, c, pl.ds(s * ROWS_T, ROWS_T)])
    pltpu.sync_copy(cnt_v, cnts_hbm.at[e, c, s])

    @pl.when(s == NS - 1)
    def _():
      pltpu.sync_copy(acc_sh.at[pl.ds(NS * ROWS_T, REM)],
                      sums_hbm.at[e, c, pl.ds(NS * ROWS_T, REM)])

    plsc.subcore_barrier()


_sc_call = pl.kernel(
    _sc_body,
    out_type=[
        jax.ShapeDtypeStruct((3, NC, N, D), jnp.float32),
        jax.ShapeDtypeStruct((3, NC, NS, NACC), jnp.float32),
    ],
    mesh=plsc.VectorSubcoreMesh(core_axis_name="c", subcore_axis_name="s"),
    compiler_params=pltpu.CompilerParams(needs_layout_passes=False),
    scratch_types=[
        pltpu.VMEM_SHARED((NACC, D), jnp.float32),
        pltpu.VMEM((LANES, D), jnp.float32),
        pltpu.VMEM((SUPER, LANES), jnp.int32),
        pltpu.VMEM((SUPER, LANES), jnp.int32),
        pltpu.VMEM((NACC,), jnp.float32),
        pltpu.SemaphoreType.DMA,
    ],
)


BLK = 1000


def _tc_body(sums_ref, cnt_ref, w_ref, b_ref, out_ref):
  acc = None
  for e in range(3):
    ssum = sums_ref[2 * e] + sums_ref[2 * e + 1]
    cnt = jnp.sum(cnt_ref[e], axis=1, keepdims=True)
    scale = 1.0 / jnp.maximum(cnt, 1.0)
    mask = (cnt > 0.5).astype(jnp.float32)
    term = jnp.dot(ssum * scale, w_ref[e], preferred_element_type=jnp.float32)
    term = term + mask * b_ref[e]
    acc = term if acc is None else acc + term
  out_ref[...] = jnp.maximum(acc, 0.0)


def _tc_call(sums, cnts, ws, bs):
  return pl.pallas_call(
      _tc_body,
      grid=(N // BLK,),
      in_specs=[
          pl.BlockSpec((6, BLK, D), lambda i: (0, i, 0)),
          pl.BlockSpec((3, BLK, NW), lambda i: (0, i, 0)),
          pl.BlockSpec((3, D, D), lambda i: (0, 0, 0)),
          pl.BlockSpec((3, 1, D), lambda i: (0, 0, 0)),
      ],
      out_specs=pl.BlockSpec((BLK, D), lambda i: (i, 0)),
      out_shape=jax.ShapeDtypeStruct((N, D), jnp.float32),
  )(sums, cnts, ws, bs)


@jax.jit
def kernel(features, edge_index_e0, edge_index_e1, edge_index_e2,
           W_e0, b_e0, W_e1, b_e1, W_e2, b_e2):
  feat_pad = jnp.concatenate(
      [features, jnp.zeros((NPAD - N, D), jnp.float32)], axis=0)
  pad = jnp.full((EPAD - E,), N, dtype=jnp.int32)
  srcs, dsts = [], []
  for ei in (edge_index_e0, edge_index_e1, edge_index_e2):
    srcs.append(jnp.concatenate([ei[0], pad]).reshape(ROWS_ALL, LANES))
    dsts.append(jnp.concatenate([ei[1], pad]).reshape(ROWS_ALL, LANES))
  src_all = jnp.stack(srcs)
  dst_all = jnp.stack(dsts)
  zc_h = jnp.zeros((NACC,), jnp.float32)

  sums, cnts = _sc_call(feat_pad, src_all, dst_all, zc_h)

  cnt_t = jnp.transpose(cnts.reshape(3, NW, NACC), (0, 2, 1))[:, :N, :]
  ws = jnp.stack([W_e0, W_e1, W_e2])
  bs = jnp.stack([b_e0, b_e1, b_e2]).reshape(3, 1, D)
  return _tc_call(sums.reshape(3 * NC, N, D), cnt_t, ws, bs)


# speedup vs baseline: 3.6421x; 1.2222x over previous
import jax
import jax.numpy as jnp
from jax import lax
from jax.experimental import pallas as pl
from jax.experimental.pallas import tpu as pltpu
from jax.experimental.pallas import tpu_sc as plsc

N = 10000
D = 128
E = 320000
NC = 2
NS = 16
NW = NC * NS
EW = 64
EPAD = 327680
ROWS_ALL = EPAD // EW
ROWS_W = ROWS_ALL // NW
SUPER = 16
NSUPER = ROWS_W // SUPER
NPAD = N + 208
NACC = N + 16
ROWS_T = 624
REM = N - NS * ROWS_T
ZROWS = 208


def _sc_body(feat_hbm, src_hbm, dst_hbm, zc_hbm,
             sums_hbm, cnts_hbm,
             acc_sh, rows_v, sidx_v, didx_v, cnt_v, gsem, ssem):
  c = lax.axis_index("c")
  s = lax.axis_index("s")
  wid = c * NS + s
  ones16 = jnp.full((16,), 1.0, jnp.float32)

  for e in range(3):
    for k in range(3):
      pltpu.sync_copy(feat_hbm.at[pl.ds(N, ZROWS)],
                      acc_sh.at[pl.ds(s * ROWS_T + k * ZROWS, ZROWS)])

    @pl.when(s == NS - 1)
    def _():
      pltpu.sync_copy(feat_hbm.at[pl.ds(N, REM)],
                      acc_sh.at[pl.ds(NS * ROWS_T, REM)])

    pltpu.sync_copy(zc_hbm, cnt_v)
    plsc.subcore_barrier()

    def superchunk(g, _):
      rowbase = wid * ROWS_W + g * SUPER
      pltpu.sync_copy(src_hbm.at[e, pl.ds(rowbase, SUPER)], sidx_v)
      pltpu.sync_copy(dst_hbm.at[e, pl.ds(rowbase, SUPER)], didx_v)
      gd = [None] * SUPER
      sd = [None] * SUPER
      gd[0] = pltpu.async_copy(feat_hbm.at[sidx_v.at[0]], rows_v.at[0], gsem)
      gd[1] = pltpu.async_copy(feat_hbm.at[sidx_v.at[1]], rows_v.at[1], gsem)
      for j in range(SUPER):
        b = j % 2
        gd[j].wait()
        sd[j] = pltpu.async_copy(rows_v.at[b], acc_sh.at[didx_v.at[j]],
                                 ssem, add=True)
        for k in range(EW // 16):
          idx = didx_v[j, pl.ds(k * 16, 16)]
          plsc.addupdate_scatter(cnt_v, [idx], ones16)
        if j + 2 < SUPER:
          sd[j].wait()
          gd[j + 2] = pltpu.async_copy(feat_hbm.at[sidx_v.at[j + 2]],
                                       rows_v.at[b], gsem)
      sd[SUPER - 2].wait()
      sd[SUPER - 1].wait()
      return 0

    lax.fori_loop(0, NSUPER, superchunk, 0)
    plsc.subcore_barrier()

    pltpu.sync_copy(acc_sh.at[pl.ds(s * ROWS_T, ROWS_T)],
                    sums_hbm.at[e, c, pl.ds(s * ROWS_T, ROWS_T)])
    pltpu.sync_copy(cnt_v, cnts_hbm.at[e, c, s])

    @pl.when(s == NS - 1)
    def _():
      pltpu.sync_copy(acc_sh.at[pl.ds(NS * ROWS_T, REM)],
                      sums_hbm.at[e, c, pl.ds(NS * ROWS_T, REM)])

    plsc.subcore_barrier()


_sc_call = pl.kernel(
    _sc_body,
    out_type=[
        jax.ShapeDtypeStruct((3, NC, N, D), jnp.float32),
        jax.ShapeDtypeStruct((3, NC, NS, NACC), jnp.float32),
    ],
    mesh=plsc.VectorSubcoreMesh(core_axis_name="c", subcore_axis_name="s"),
    compiler_params=pltpu.CompilerParams(needs_layout_passes=False),
    scratch_types=[
        pltpu.VMEM_SHARED((NACC, D), jnp.float32),
        pltpu.VMEM((2, EW, D), jnp.float32),
        pltpu.VMEM((SUPER, EW), jnp.int32),
        pltpu.VMEM((SUPER, EW), jnp.int32),
        pltpu.VMEM((NACC,), jnp.float32),
        pltpu.SemaphoreType.DMA,
        pltpu.SemaphoreType.DMA,
    ],
)


BLK = 1000


def _tc_body(sums_ref, cnt_ref, w_ref, b_ref, out_ref):
  acc = None
  for e in range(3):
    ssum = sums_ref[2 * e] + sums_ref[2 * e + 1]
    cnt = jnp.sum(cnt_ref[e], axis=1, keepdims=True)
    scale = 1.0 / jnp.maximum(cnt, 1.0)
    mask = (cnt > 0.5).astype(jnp.float32)
    term = jnp.dot(ssum * scale, w_ref[e], preferred_element_type=jnp.float32)
    term = term + mask * b_ref[e]
    acc = term if acc is None else acc + term
  out_ref[...] = jnp.maximum(acc, 0.0)


def _tc_call(sums, cnts, ws, bs):
  return pl.pallas_call(
      _tc_body,
      grid=(N // BLK,),
      in_specs=[
          pl.BlockSpec((6, BLK, D), lambda i: (0, i, 0)),
          pl.BlockSpec((3, BLK, NW), lambda i: (0, i, 0)),
          pl.BlockSpec((3, D, D), lambda i: (0, 0, 0)),
          pl.BlockSpec((3, 1, D), lambda i: (0, 0, 0)),
      ],
      out_specs=pl.BlockSpec((BLK, D), lambda i: (i, 0)),
      out_shape=jax.ShapeDtypeStruct((N, D), jnp.float32),
  )(sums, cnts, ws, bs)


@jax.jit
def kernel(features, edge_index_e0, edge_index_e1, edge_index_e2,
           W_e0, b_e0, W_e1, b_e1, W_e2, b_e2):
  feat_pad = jnp.concatenate(
      [features, jnp.zeros((NPAD - N, D), jnp.float32)], axis=0)
  pad = jnp.full((EPAD - E,), N, dtype=jnp.int32)
  srcs, dsts = [], []
  for ei in (edge_index_e0, edge_index_e1, edge_index_e2):
    srcs.append(jnp.concatenate([ei[0], pad]).reshape(ROWS_ALL, EW))
    dsts.append(jnp.concatenate([ei[1], pad]).reshape(ROWS_ALL, EW))
  src_all = jnp.stack(srcs)
  dst_all = jnp.stack(dsts)
  zc_h = jnp.zeros((NACC,), jnp.float32)

  sums, cnts = _sc_call(feat_pad, src_all, dst_all, zc_h)

  cnt_t = jnp.transpose(cnts.reshape(3, NW, NACC), (0, 2, 1))[:, :N, :]
  ws = jnp.stack([W_e0, W_e1, W_e2])
  bs = jnp.stack([b_e0, b_e1, b_e2]).reshape(3, 1, D)
  return _tc_call(sums.reshape(3 * NC, N, D), cnt_t, ws, bs)


# speedup vs baseline: 8.9051x; 2.4450x over previous
import jax
import jax.numpy as jnp
from jax import lax
from jax.experimental import pallas as pl
from jax.experimental.pallas import tpu as pltpu
from jax.experimental.pallas import tpu_sc as plsc

N = 10000
D = 128
E = 320000
NC = 2
NS = 16
NW = NC * NS
EW = 64
EPAD = 327680
ROWS_ALL = EPAD // EW
ROWS_W = ROWS_ALL // NW
SUPER = 16
NSUPER = ROWS_W // SUPER
NPAD = N + 208
NACC = N
PADE = EPAD - E
ROWS_T = 624
REM = N - NS * ROWS_T
ZROWS = 208


def _sc_body(feat_hbm, src_hbm, dst_hbm, zc_hbm,
             sums_hbm, cnts_hbm,
             acc_sh, rows_v, sidx_v, didx_v, cnt_v, gsem, ssem):
  c = lax.axis_index("c")
  s = lax.axis_index("s")
  wid = c * NS + s
  ones16 = jnp.full((16,), 1.0, jnp.float32)

  for e in range(3):
    for k in range(3):
      pltpu.sync_copy(feat_hbm.at[pl.ds(N, ZROWS)],
                      acc_sh.at[pl.ds(s * ROWS_T + k * ZROWS, ZROWS)])

    @pl.when(s == NS - 1)
    def _():
      pltpu.sync_copy(feat_hbm.at[pl.ds(N, REM)],
                      acc_sh.at[pl.ds(NS * ROWS_T, REM)])

    pltpu.sync_copy(zc_hbm, cnt_v)
    plsc.subcore_barrier()

    def superchunk(g, _):
      rowbase = wid * ROWS_W + g * SUPER
      pltpu.sync_copy(src_hbm.at[e, pl.ds(rowbase, SUPER)], sidx_v)
      pltpu.sync_copy(dst_hbm.at[e, pl.ds(rowbase, SUPER)], didx_v)
      gd = [None] * SUPER
      sd = [None] * SUPER
      gd[0] = pltpu.async_copy(feat_hbm.at[sidx_v.at[0]], rows_v.at[0], gsem)
      gd[1] = pltpu.async_copy(feat_hbm.at[sidx_v.at[1]], rows_v.at[1], gsem)
      for j in range(SUPER):
        b = j % 2
        gd[j].wait()
        sd[j] = pltpu.async_copy(rows_v.at[b], acc_sh.at[didx_v.at[j]],
                                 ssem, add=True)
        for k in range(EW // 16):
          idx = didx_v[j, pl.ds(k * 16, 16)]
          plsc.addupdate_scatter(cnt_v, [idx], ones16)
        if j + 2 < SUPER:
          sd[j].wait()
          gd[j + 2] = pltpu.async_copy(feat_hbm.at[sidx_v.at[j + 2]],
                                       rows_v.at[b], gsem)
      sd[SUPER - 2].wait()
      sd[SUPER - 1].wait()
      return 0

    lax.fori_loop(0, NSUPER, superchunk, 0)
    plsc.subcore_barrier()

    pltpu.sync_copy(acc_sh.at[pl.ds(s * ROWS_T, ROWS_T)],
                    sums_hbm.at[e, c, pl.ds(s * ROWS_T, ROWS_T)])
    pltpu.sync_copy(cnt_v, cnts_hbm.at[e, c, s])

    @pl.when(s == NS - 1)
    def _():
      pltpu.sync_copy(acc_sh.at[pl.ds(NS * ROWS_T, REM)],
                      sums_hbm.at[e, c, pl.ds(NS * ROWS_T, REM)])

    plsc.subcore_barrier()


_sc_call = pl.kernel(
    _sc_body,
    out_type=[
        jax.ShapeDtypeStruct((3, NC, N, D), jnp.float32),
        jax.ShapeDtypeStruct((3, NC, NS, NACC), jnp.float32),
    ],
    mesh=plsc.VectorSubcoreMesh(core_axis_name="c", subcore_axis_name="s"),
    compiler_params=pltpu.CompilerParams(needs_layout_passes=False),
    scratch_types=[
        pltpu.VMEM_SHARED((NACC, D), jnp.float32),
        pltpu.VMEM((2, EW, D), jnp.float32),
        pltpu.VMEM((SUPER, EW), jnp.int32),
        pltpu.VMEM((SUPER, EW), jnp.int32),
        pltpu.VMEM((NACC,), jnp.float32),
        pltpu.SemaphoreType.DMA,
        pltpu.SemaphoreType.DMA,
    ],
)


BLK = 1000


def _tc_body(sums_ref, cnt_ref, w_ref, b_ref, out_ref):
  i = pl.program_id(0)
  rowid = lax.broadcasted_iota(jnp.int32, (BLK, 1), 0) + i * BLK
  padc = (rowid < PADE).astype(jnp.float32)
  acc = None
  for e in range(3):
    ssum = sums_ref[2 * e] + sums_ref[2 * e + 1]
    cnt = jnp.sum(cnt_ref[e], axis=1, keepdims=True) - padc
    scale = 1.0 / jnp.maximum(cnt, 1.0)
    mask = (cnt > 0.5).astype(jnp.float32)
    term = jnp.dot(ssum * scale, w_ref[e], preferred_element_type=jnp.float32)
    term = term + mask * b_ref[e]
    acc = term if acc is None else acc + term
  out_ref[...] = jnp.maximum(acc, 0.0)


def _tc_call(sums, cnts, ws, bs):
  return pl.pallas_call(
      _tc_body,
      grid=(N // BLK,),
      in_specs=[
          pl.BlockSpec((6, BLK, D), lambda i: (0, i, 0)),
          pl.BlockSpec((3, BLK, NW), lambda i: (0, i, 0)),
          pl.BlockSpec((3, D, D), lambda i: (0, 0, 0)),
          pl.BlockSpec((3, 1, D), lambda i: (0, 0, 0)),
      ],
      out_specs=pl.BlockSpec((BLK, D), lambda i: (i, 0)),
      out_shape=jax.ShapeDtypeStruct((N, D), jnp.float32),
  )(sums, cnts, ws, bs)


@jax.jit
def kernel(features, edge_index_e0, edge_index_e1, edge_index_e2,
           W_e0, b_e0, W_e1, b_e1, W_e2, b_e2):
  feat_pad = jnp.concatenate(
      [features, jnp.zeros((NPAD - N, D), jnp.float32)], axis=0)
  pad_src = N + (jnp.arange(PADE, dtype=jnp.int32) % (NPAD - N))
  pad_dst = jnp.arange(PADE, dtype=jnp.int32)
  srcs, dsts = [], []
  for ei in (edge_index_e0, edge_index_e1, edge_index_e2):
    srcs.append(jnp.concatenate([ei[0], pad_src]).reshape(ROWS_ALL, EW))
    dsts.append(jnp.concatenate([ei[1], pad_dst]).reshape(ROWS_ALL, EW))
  src_all = jnp.stack(srcs)
  dst_all = jnp.stack(dsts)
  zc_h = jnp.zeros((NACC,), jnp.float32)

  sums, cnts = _sc_call(feat_pad, src_all, dst_all, zc_h)

  cnt_t = jnp.transpose(cnts.reshape(3, NW, NACC), (0, 2, 1))[:, :N, :]
  ws = jnp.stack([W_e0, W_e1, W_e2])
  bs = jnp.stack([b_e0, b_e1, b_e2]).reshape(3, 1, D)
  return _tc_call(sums.reshape(3 * NC, N, D), cnt_t, ws, bs)


# speedup vs baseline: 9.4665x; 1.0630x over previous
import jax
import jax.numpy as jnp
from jax import lax
from jax.experimental import pallas as pl
from jax.experimental.pallas import tpu as pltpu
from jax.experimental.pallas import tpu_sc as plsc

N = 10000
D = 128
E = 320000
NC = 2
NS = 16
NW = NC * NS
EW = 64
EPAD = 327680
ROWS_ALL = EPAD // EW
ROWS_W = ROWS_ALL // NW
SUPER = 16
NSUPER = ROWS_W // SUPER
NPAD = N + 208
NACC = N
PADE = EPAD - E
ROWS_T = 624
REM = N - NS * ROWS_T
ZROWS = 208


def _sc_body(feat_hbm, src_hbm, dst_hbm, zc_hbm,
             sums_hbm, cnts_hbm,
             acc_sh, rows_v, sidx_v, didx_v, cnt_v, gsem, ssem, isem0, isem1):
  c = lax.axis_index("c")
  s = lax.axis_index("s")
  wid = c * NS + s
  ones16 = jnp.full((16,), 1.0, jnp.float32)

  for e in range(3):
    for k in range(3):
      pltpu.sync_copy(feat_hbm.at[pl.ds(N, ZROWS)],
                      acc_sh.at[pl.ds(s * ROWS_T + k * ZROWS, ZROWS)])

    @pl.when(s == NS - 1)
    def _():
      pltpu.sync_copy(feat_hbm.at[pl.ds(N, REM)],
                      acc_sh.at[pl.ds(NS * ROWS_T, REM)])

    pltpu.sync_copy(zc_hbm, cnt_v)

    pltpu.async_copy(src_hbm.at[e, pl.ds(wid * ROWS_W, SUPER)],
                     sidx_v.at[0], isem0.at[0])
    pltpu.async_copy(dst_hbm.at[e, pl.ds(wid * ROWS_W, SUPER)],
                     didx_v.at[0], isem1.at[0])
    plsc.subcore_barrier()

    def superchunk(g, _):
      rowbase = wid * ROWS_W + g * SUPER
      pg = lax.rem(g, 2)
      png = 1 - pg
      @pl.when(g + 1 < NSUPER)
      def _():
        nxt = rowbase + SUPER
        pltpu.async_copy(src_hbm.at[e, pl.ds(nxt, SUPER)],
                         sidx_v.at[png], isem0.at[png])
        pltpu.async_copy(dst_hbm.at[e, pl.ds(nxt, SUPER)],
                         didx_v.at[png], isem1.at[png])

      pltpu.make_async_copy(src_hbm.at[e, pl.ds(rowbase, SUPER)],
                            sidx_v.at[pg], isem0.at[pg]).wait()
      pltpu.make_async_copy(dst_hbm.at[e, pl.ds(rowbase, SUPER)],
                            didx_v.at[pg], isem1.at[pg]).wait()

      gd = [None] * SUPER
      sd = [None] * SUPER
      gd[0] = pltpu.async_copy(feat_hbm.at[sidx_v.at[pg, 0]], rows_v.at[0],
                               gsem)
      gd[1] = pltpu.async_copy(feat_hbm.at[sidx_v.at[pg, 1]], rows_v.at[1],
                               gsem)
      for j in range(SUPER):
        b = j % 2
        gd[j].wait()
        sd[j] = pltpu.async_copy(rows_v.at[b], acc_sh.at[didx_v.at[pg, j]],
                                 ssem, add=True)
        for k in range(EW // 16):
          idx = didx_v[pg, j, pl.ds(k * 16, 16)]
          plsc.addupdate_scatter(cnt_v, [idx], ones16)
        if j + 2 < SUPER:
          sd[j].wait()
          gd[j + 2] = pltpu.async_copy(feat_hbm.at[sidx_v.at[pg, j + 2]],
                                       rows_v.at[b], gsem)
      sd[SUPER - 2].wait()
      sd[SUPER - 1].wait()
      return 0

    lax.fori_loop(0, NSUPER, superchunk, 0)
    plsc.subcore_barrier()

    pltpu.sync_copy(acc_sh.at[pl.ds(s * ROWS_T, ROWS_T)],
                    sums_hbm.at[e, c, pl.ds(s * ROWS_T, ROWS_T)])
    pltpu.sync_copy(cnt_v, cnts_hbm.at[e, c, s])

    @pl.when(s == NS - 1)
    def _():
      pltpu.sync_copy(acc_sh.at[pl.ds(NS * ROWS_T, REM)],
                      sums_hbm.at[e, c, pl.ds(NS * ROWS_T, REM)])

    plsc.subcore_barrier()


_sc_call = pl.kernel(
    _sc_body,
    out_type=[
        jax.ShapeDtypeStruct((3, NC, N, D), jnp.float32),
        jax.ShapeDtypeStruct((3, NC, NS, NACC), jnp.float32),
    ],
    mesh=plsc.VectorSubcoreMesh(core_axis_name="c", subcore_axis_name="s"),
    compiler_params=pltpu.CompilerParams(needs_layout_passes=False),
    scratch_types=[
        pltpu.VMEM_SHARED((NACC, D), jnp.float32),
        pltpu.VMEM((2, EW, D), jnp.float32),
        pltpu.VMEM((2, SUPER, EW), jnp.int32),
        pltpu.VMEM((2, SUPER, EW), jnp.int32),
        pltpu.VMEM((NACC,), jnp.float32),
        pltpu.SemaphoreType.DMA,
        pltpu.SemaphoreType.DMA,
        pltpu.SemaphoreType.DMA((2,)),
        pltpu.SemaphoreType.DMA((2,)),
    ],
)


BLK = 1000


def _tc_body(sums_ref, cnt_ref, w_ref, b_ref, out_ref):
  i = pl.program_id(0)
  rowid = lax.broadcasted_iota(jnp.int32, (BLK, 1), 0) + i * BLK
  padc = (rowid < PADE).astype(jnp.float32)
  acc = None
  for e in range(3):
    ssum = sums_ref[2 * e] + sums_ref[2 * e + 1]
    cnt = jnp.sum(cnt_ref[e], axis=1, keepdims=True) - padc
    scale = 1.0 / jnp.maximum(cnt, 1.0)
    mask = (cnt > 0.5).astype(jnp.float32)
    term = jnp.dot(ssum * scale, w_ref[e], preferred_element_type=jnp.float32)
    term = term + mask * b_ref[e]
    acc = term if acc is None else acc + term
  out_ref[...] = jnp.maximum(acc, 0.0)


def _tc_call(sums, cnts, ws, bs):
  return pl.pallas_call(
      _tc_body,
      grid=(N // BLK,),
      in_specs=[
          pl.BlockSpec((6, BLK, D), lambda i: (0, i, 0)),
          pl.BlockSpec((3, BLK, NW), lambda i: (0, i, 0)),
          pl.BlockSpec((3, D, D), lambda i: (0, 0, 0)),
          pl.BlockSpec((3, 1, D), lambda i: (0, 0, 0)),
      ],
      out_specs=pl.BlockSpec((BLK, D), lambda i: (i, 0)),
      out_shape=jax.ShapeDtypeStruct((N, D), jnp.float32),
  )(sums, cnts, ws, bs)


@jax.jit
def kernel(features, edge_index_e0, edge_index_e1, edge_index_e2,
           W_e0, b_e0, W_e1, b_e1, W_e2, b_e2):
  feat_pad = jnp.concatenate(
      [features, jnp.zeros((NPAD - N, D), jnp.float32)], axis=0)
  pad_src = N + (jnp.arange(PADE, dtype=jnp.int32) % (NPAD - N))
  pad_dst = jnp.arange(PADE, dtype=jnp.int32)
  srcs, dsts = [], []
  for ei in (edge_index_e0, edge_index_e1, edge_index_e2):
    srcs.append(jnp.concatenate([ei[0], pad_src]).reshape(ROWS_ALL, EW))
    dsts.append(jnp.concatenate([ei[1], pad_dst]).reshape(ROWS_ALL, EW))
  src_all = jnp.stack(srcs)
  dst_all = jnp.stack(dsts)
  zc_h = jnp.zeros((NACC,), jnp.float32)

  sums, cnts = _sc_call(feat_pad, src_all, dst_all, zc_h)

  cnt_t = jnp.transpose(cnts.reshape(3, NW, NACC), (0, 2, 1))[:, :N, :]
  ws = jnp.stack([W_e0, W_e1, W_e2])
  bs = jnp.stack([b_e0, b_e1, b_e2]).reshape(3, 1, D)
  return _tc_call(sums.reshape(3 * NC, N, D), cnt_t, ws, bs)


# speedup vs baseline: 9.8447x; 1.0400x over previous
import jax
import jax.numpy as jnp
from jax import lax
from jax.experimental import pallas as pl
from jax.experimental.pallas import tpu as pltpu
from jax.experimental.pallas import tpu_sc as plsc

N = 10000
D = 128
E = 320000
NC = 2
NS = 16
NW = NC * NS
EW = 64
EPAD = 327680
ROWS_ALL = EPAD // EW
ROWS_W = ROWS_ALL // NW
SUPER = 16
NSUPER = ROWS_W // SUPER
NPAD = N + 208
NACC = N
PADE = EPAD - E
ROWS_T = 624
REM = N - NS * ROWS_T
ZROWS = 208


def _sc_body(feat_hbm, src_hbm, dst_hbm, zc_hbm,
             sums_hbm, cnts_hbm,
             acc_sh, rows_v, sidx_v, didx_v, cnt_v, gsem, ssem, isem0, isem1):
  c = lax.axis_index("c")
  s = lax.axis_index("s")
  wid = c * NS + s
  ones16 = jnp.full((16,), 1.0, jnp.float32)

  for p in range(2):
    if p == 0:
      et = c
      nsuper = 2 * NSUPER
      rowstart = s * (2 * ROWS_W)
    else:
      et = 2
      nsuper = NSUPER
      rowstart = c * (ROWS_ALL // 2) + s * ROWS_W

    for k in range(3):
      pltpu.sync_copy(feat_hbm.at[pl.ds(N, ZROWS)],
                      acc_sh.at[pl.ds(s * ROWS_T + k * ZROWS, ZROWS)])

    @pl.when(s == NS - 1)
    def _():
      pltpu.sync_copy(feat_hbm.at[pl.ds(N, REM)],
                      acc_sh.at[pl.ds(NS * ROWS_T, REM)])

    pltpu.sync_copy(zc_hbm, cnt_v)

    pltpu.async_copy(src_hbm.at[et, pl.ds(rowstart, SUPER)],
                     sidx_v.at[0], isem0.at[0])
    pltpu.async_copy(dst_hbm.at[et, pl.ds(rowstart, SUPER)],
                     didx_v.at[0], isem1.at[0])
    plsc.subcore_barrier()

    def superchunk(g, _):
      rowbase = rowstart + g * SUPER
      pg = lax.rem(g, 2)
      png = 1 - pg
      @pl.when(g + 1 < nsuper)
      def _():
        nxt = rowbase + SUPER
        pltpu.async_copy(src_hbm.at[et, pl.ds(nxt, SUPER)],
                         sidx_v.at[png], isem0.at[png])
        pltpu.async_copy(dst_hbm.at[et, pl.ds(nxt, SUPER)],
                         didx_v.at[png], isem1.at[png])

      pltpu.make_async_copy(src_hbm.at[et, pl.ds(rowbase, SUPER)],
                            sidx_v.at[pg], isem0.at[pg]).wait()
      pltpu.make_async_copy(dst_hbm.at[et, pl.ds(rowbase, SUPER)],
                            didx_v.at[pg], isem1.at[pg]).wait()

      gd = [None] * SUPER
      sd = [None] * SUPER
      gd[0] = pltpu.async_copy(feat_hbm.at[sidx_v.at[pg, 0]], rows_v.at[0],
                               gsem)
      gd[1] = pltpu.async_copy(feat_hbm.at[sidx_v.at[pg, 1]], rows_v.at[1],
                               gsem)
      for j in range(SUPER):
        b = j % 2
        gd[j].wait()
        sd[j] = pltpu.async_copy(rows_v.at[b], acc_sh.at[didx_v.at[pg, j]],
                                 ssem, add=True)
        for k in range(EW // 16):
          idx = didx_v[pg, j, pl.ds(k * 16, 16)]
          plsc.addupdate_scatter(cnt_v, [idx], ones16)
        if j + 2 < SUPER:
          sd[j].wait()
          gd[j + 2] = pltpu.async_copy(feat_hbm.at[sidx_v.at[pg, j + 2]],
                                       rows_v.at[b], gsem)
      sd[SUPER - 2].wait()
      sd[SUPER - 1].wait()
      return 0

    lax.fori_loop(0, nsuper, superchunk, 0)
    plsc.subcore_barrier()

    pltpu.sync_copy(acc_sh.at[pl.ds(s * ROWS_T, ROWS_T)],
                    sums_hbm.at[p, c, pl.ds(s * ROWS_T, ROWS_T)])
    pltpu.sync_copy(cnt_v, cnts_hbm.at[p, c, s])

    @pl.when(s == NS - 1)
    def _():
      pltpu.sync_copy(acc_sh.at[pl.ds(NS * ROWS_T, REM)],
                      sums_hbm.at[p, c, pl.ds(NS * ROWS_T, REM)])

    plsc.subcore_barrier()


_sc_call = pl.kernel(
    _sc_body,
    out_type=[
        jax.ShapeDtypeStruct((2, NC, N, D), jnp.float32),
        jax.ShapeDtypeStruct((2, NC, NS, NACC), jnp.float32),
    ],
    mesh=plsc.VectorSubcoreMesh(core_axis_name="c", subcore_axis_name="s"),
    compiler_params=pltpu.CompilerParams(needs_layout_passes=False),
    scratch_types=[
        pltpu.VMEM_SHARED((NACC, D), jnp.float32),
        pltpu.VMEM((2, EW, D), jnp.float32),
        pltpu.VMEM((2, SUPER, EW), jnp.int32),
        pltpu.VMEM((2, SUPER, EW), jnp.int32),
        pltpu.VMEM((NACC,), jnp.float32),
        pltpu.SemaphoreType.DMA,
        pltpu.SemaphoreType.DMA,
        pltpu.SemaphoreType.DMA((2,)),
        pltpu.SemaphoreType.DMA((2,)),
    ],
)


BLK = 1000


def _tc_body(sums_ref, cnt_ref, w_ref, b_ref, out_ref):
  i = pl.program_id(0)
  rowid = lax.broadcasted_iota(jnp.int32, (BLK, 1), 0) + i * BLK
  padc = (rowid < PADE).astype(jnp.float32)
  acc = None
  for e in range(3):
    if e < 2:
      ssum = sums_ref[e]
      cnt = jnp.sum(cnt_ref[e], axis=1, keepdims=True) - padc
    else:
      ssum = sums_ref[2] + sums_ref[3]
      cnt = (jnp.sum(cnt_ref[2], axis=1, keepdims=True)
             + jnp.sum(cnt_ref[3], axis=1, keepdims=True) - padc)
    scale = 1.0 / jnp.maximum(cnt, 1.0)
    mask = (cnt > 0.5).astype(jnp.float32)
    term = jnp.dot(ssum * scale, w_ref[e], preferred_element_type=jnp.float32)
    term = term + mask * b_ref[e]
    acc = term if acc is None else acc + term
  out_ref[...] = jnp.maximum(acc, 0.0)


def _tc_call(sums, cnts, ws, bs):
  return pl.pallas_call(
      _tc_body,
      grid=(N // BLK,),
      in_specs=[
          pl.BlockSpec((4, BLK, D), lambda i: (0, i, 0)),
          pl.BlockSpec((4, BLK, NS), lambda i: (0, i, 0)),
          pl.BlockSpec((3, D, D), lambda i: (0, 0, 0)),
          pl.BlockSpec((3, 1, D), lambda i: (0, 0, 0)),
      ],
      out_specs=pl.BlockSpec((BLK, D), lambda i: (i, 0)),
      out_shape=jax.ShapeDtypeStruct((N, D), jnp.float32),
  )(sums, cnts, ws, bs)


@jax.jit
def kernel(features, edge_index_e0, edge_index_e1, edge_index_e2,
           W_e0, b_e0, W_e1, b_e1, W_e2, b_e2):
  feat_pad = jnp.concatenate(
      [features, jnp.zeros((NPAD - N, D), jnp.float32)], axis=0)
  pad_src = N + (jnp.arange(PADE, dtype=jnp.int32) % (NPAD - N))
  pad_dst = jnp.arange(PADE, dtype=jnp.int32)
  srcs, dsts = [], []
  for ei in (edge_index_e0, edge_index_e1, edge_index_e2):
    srcs.append(jnp.concatenate([ei[0], pad_src]).reshape(ROWS_ALL, EW))
    dsts.append(jnp.concatenate([ei[1], pad_dst]).reshape(ROWS_ALL, EW))
  src_all = jnp.stack(srcs)
  dst_all = jnp.stack(dsts)
  zc_h = jnp.zeros((NACC,), jnp.float32)

  sums, cnts = _sc_call(feat_pad, src_all, dst_all, zc_h)

  cnt_t = jnp.transpose(cnts.reshape(2 * NC, NS, NACC), (0, 2, 1))[:, :N, :]
  ws = jnp.stack([W_e0, W_e1, W_e2])
  bs = jnp.stack([b_e0, b_e1, b_e2]).reshape(3, 1, D)
  return _tc_call(sums.reshape(2 * NC, N, D), cnt_t, ws, bs)


# speedup vs baseline: 11.0331x; 1.1207x over previous
import jax
import jax.numpy as jnp
from jax import lax
from jax.experimental import pallas as pl
from jax.experimental.pallas import tpu as pltpu
from jax.experimental.pallas import tpu_sc as plsc

N = 10000
D = 128
E = 320000
NC = 2
NS = 16
NW = NC * NS
EW = 64
EPAD = 327680
ROWS_ALL = EPAD // EW
ROWS_W = ROWS_ALL // NW
SUPER = 8
NSUPER = ROWS_W // SUPER
NPAD = N + 208
NACC = N
PADE = EPAD - E
ROWS_T = 624
REM = N - NS * ROWS_T
ZROWS = 208


def _sc_body(feat_hbm, src_hbm, dst_hbm, zc_hbm,
             sums_hbm, cnts_hbm,
             acc_sh, rows_v, sidx_v, didx_v, cnt_v, gsem, ssem, isem0, isem1):
  c = lax.axis_index("c")
  s = lax.axis_index("s")
  wid = c * NS + s
  ones16 = jnp.full((16,), 1.0, jnp.float32)

  for p in range(2):
    if p == 0:
      et = c
      nsuper = 2 * NSUPER
      rowstart = s * (2 * ROWS_W)
    else:
      et = 2
      nsuper = NSUPER
      rowstart = c * (ROWS_ALL // 2) + s * ROWS_W

    for k in range(3):
      pltpu.sync_copy(feat_hbm.at[pl.ds(N, ZROWS)],
                      acc_sh.at[pl.ds(s * ROWS_T + k * ZROWS, ZROWS)])

    @pl.when(s == NS - 1)
    def _():
      pltpu.sync_copy(feat_hbm.at[pl.ds(N, REM)],
                      acc_sh.at[pl.ds(NS * ROWS_T, REM)])

    pltpu.sync_copy(zc_hbm, cnt_v)

    pltpu.async_copy(src_hbm.at[et, pl.ds(rowstart, SUPER)],
                     sidx_v.at[0], isem0.at[0])
    pltpu.async_copy(dst_hbm.at[et, pl.ds(rowstart, SUPER)],
                     didx_v.at[0], isem1.at[0])
    plsc.subcore_barrier()

    def superchunk(g, _):
      rowbase = rowstart + g * SUPER
      pg = lax.rem(g, 2)
      png = 1 - pg
      @pl.when(g + 1 < nsuper)
      def _():
        nxt = rowbase + SUPER
        pltpu.async_copy(src_hbm.at[et, pl.ds(nxt, SUPER)],
                         sidx_v.at[png], isem0.at[png])
        pltpu.async_copy(dst_hbm.at[et, pl.ds(nxt, SUPER)],
                         didx_v.at[png], isem1.at[png])

      pltpu.make_async_copy(src_hbm.at[et, pl.ds(rowbase, SUPER)],
                            sidx_v.at[pg], isem0.at[pg]).wait()
      pltpu.make_async_copy(dst_hbm.at[et, pl.ds(rowbase, SUPER)],
                            didx_v.at[pg], isem1.at[pg]).wait()

      gd = [None] * SUPER
      sd = [None] * SUPER
      for j in range(3):
        gd[j] = pltpu.async_copy(feat_hbm.at[sidx_v.at[pg, j]], rows_v.at[j],
                                 gsem)
      for j in range(SUPER):
        b = j % 3
        gd[j].wait()
        sd[j] = pltpu.async_copy(rows_v.at[b], acc_sh.at[didx_v.at[pg, j]],
                                 ssem, add=True)
        for k in range(EW // 16):
          idx = didx_v[pg, j, pl.ds(k * 16, 16)]
          plsc.addupdate_scatter(cnt_v, [idx], ones16)
        if j + 3 < SUPER:
          sd[j].wait()
          gd[j + 3] = pltpu.async_copy(feat_hbm.at[sidx_v.at[pg, j + 3]],
                                       rows_v.at[b], gsem)
      for j in range(SUPER - 3, SUPER):
        sd[j].wait()
      return 0

    lax.fori_loop(0, nsuper, superchunk, 0)
    plsc.subcore_barrier()

    pltpu.sync_copy(acc_sh.at[pl.ds(s * ROWS_T, ROWS_T)],
                    sums_hbm.at[p, c, pl.ds(s * ROWS_T, ROWS_T)])
    pltpu.sync_copy(cnt_v, cnts_hbm.at[p, c, s])

    @pl.when(s == NS - 1)
    def _():
      pltpu.sync_copy(acc_sh.at[pl.ds(NS * ROWS_T, REM)],
                      sums_hbm.at[p, c, pl.ds(NS * ROWS_T, REM)])

    plsc.subcore_barrier()


_sc_call = pl.kernel(
    _sc_body,
    out_type=[
        jax.ShapeDtypeStruct((2, NC, N, D), jnp.float32),
        jax.ShapeDtypeStruct((2, NC, NS, NACC), jnp.float32),
    ],
    mesh=plsc.VectorSubcoreMesh(core_axis_name="c", subcore_axis_name="s"),
    compiler_params=pltpu.CompilerParams(needs_layout_passes=False),
    scratch_types=[
        pltpu.VMEM_SHARED((NACC, D), jnp.float32),
        pltpu.VMEM((3, EW, D), jnp.float32),
        pltpu.VMEM((2, SUPER, EW), jnp.int32),
        pltpu.VMEM((2, SUPER, EW), jnp.int32),
        pltpu.VMEM((NACC,), jnp.float32),
        pltpu.SemaphoreType.DMA,
        pltpu.SemaphoreType.DMA,
        pltpu.SemaphoreType.DMA((2,)),
        pltpu.SemaphoreType.DMA((2,)),
    ],
)


BLK = 1000


def _tc_body(sums_ref, cnt_ref, w_ref, b_ref, out_ref):
  i = pl.program_id(0)
  rowid = lax.broadcasted_iota(jnp.int32, (BLK, 1), 0) + i * BLK
  padc = (rowid < PADE).astype(jnp.float32)
  acc = None
  for e in range(3):
    if e < 2:
      ssum = sums_ref[e]
      cnt = jnp.sum(cnt_ref[e], axis=1, keepdims=True) - padc
    else:
      ssum = sums_ref[2] + sums_ref[3]
      cnt = (jnp.sum(cnt_ref[2], axis=1, keepdims=True)
             + jnp.sum(cnt_ref[3], axis=1, keepdims=True) - padc)
    scale = 1.0 / jnp.maximum(cnt, 1.0)
    mask = (cnt > 0.5).astype(jnp.float32)
    term = jnp.dot(ssum * scale, w_ref[e], preferred_element_type=jnp.float32)
    term = term + mask * b_ref[e]
    acc = term if acc is None else acc + term
  out_ref[...] = jnp.maximum(acc, 0.0)


def _tc_call(sums, cnts, ws, bs):
  return pl.pallas_call(
      _tc_body,
      grid=(N // BLK,),
      in_specs=[
          pl.BlockSpec((4, BLK, D), lambda i: (0, i, 0)),
          pl.BlockSpec((4, BLK, NS), lambda i: (0, i, 0)),
          pl.BlockSpec((3, D, D), lambda i: (0, 0, 0)),
          pl.BlockSpec((3, 1, D), lambda i: (0, 0, 0)),
      ],
      out_specs=pl.BlockSpec((BLK, D), lambda i: (i, 0)),
      out_shape=jax.ShapeDtypeStruct((N, D), jnp.float32),
  )(sums, cnts, ws, bs)


@jax.jit
def kernel(features, edge_index_e0, edge_index_e1, edge_index_e2,
           W_e0, b_e0, W_e1, b_e1, W_e2, b_e2):
  feat_pad = jnp.concatenate(
      [features, jnp.zeros((NPAD - N, D), jnp.float32)], axis=0)
  pad_src = N + (jnp.arange(PADE, dtype=jnp.int32) % (NPAD - N))
  pad_dst = jnp.arange(PADE, dtype=jnp.int32)
  srcs, dsts = [], []
  for ei in (edge_index_e0, edge_index_e1, edge_index_e2):
    srcs.append(jnp.concatenate([ei[0], pad_src]).reshape(ROWS_ALL, EW))
    dsts.append(jnp.concatenate([ei[1], pad_dst]).reshape(ROWS_ALL, EW))
  src_all = jnp.stack(srcs)
  dst_all = jnp.stack(dsts)
  zc_h = jnp.zeros((NACC,), jnp.float32)

  sums, cnts = _sc_call(feat_pad, src_all, dst_all, zc_h)

  cnt_t = jnp.transpose(cnts.reshape(2 * NC, NS, NACC), (0, 2, 1))[:, :N, :]
  ws = jnp.stack([W_e0, W_e1, W_e2])
  bs = jnp.stack([b_e0, b_e1, b_e2]).reshape(3, 1, D)
  return _tc_call(sums.reshape(2 * NC, N, D), cnt_t, ws, bs)


# speedup vs baseline: 12.4865x; 1.1317x over previous
import jax
import jax.numpy as jnp
from jax import lax
from jax.experimental import pallas as pl
from jax.experimental.pallas import tpu as pltpu
from jax.experimental.pallas import tpu_sc as plsc

N = 10000
D = 128
E = 320000
NC = 2
NS = 16
NW = NC * NS
EW = 64
ROWS_ALL = E // EW
SUPER = 8
NACC = N
ROWS_T = 624
REM = N - NS * ROWS_T
ZROWS = 208
P0_SC = ROWS_ALL // (NS * SUPER)
P1_SC = ROWS_ALL // (NW * SUPER)
P1_XTRA = ROWS_ALL // SUPER - NW * P1_SC


def _sc_body(feat_hbm, e0_hbm, e1_hbm, e2_hbm, zz_hbm, zc_hbm,
             sums_hbm, cnts_hbm,
             acc_sh, rows_v, sidx_v, didx_v, cnt_v, gsem, ssem, isem0, isem1):
  c = lax.axis_index("c")
  s = lax.axis_index("s")
  wid = c * NS + s
  ones16 = jnp.full((16,), 1.0, jnp.float32)

  def zero_phase():
    for k in range(3):
      pltpu.sync_copy(zz_hbm,
                      acc_sh.at[pl.ds(s * ROWS_T + k * ZROWS, ZROWS)])

    @pl.when(s == NS - 1)
    def _():
      pltpu.sync_copy(zz_hbm.at[pl.ds(0, REM)],
                      acc_sh.at[pl.ds(NS * ROWS_T, REM)])

    pltpu.sync_copy(zc_hbm, cnt_v)

  def accum(ei_hbm, rowstart, nsuper):
    pltpu.async_copy(ei_hbm.at[0, pl.ds(rowstart, SUPER)],
                     sidx_v.at[0], isem0.at[0])
    pltpu.async_copy(ei_hbm.at[1, pl.ds(rowstart, SUPER)],
                     didx_v.at[0], isem1.at[0])

    def superchunk(g, _):
      rowbase = rowstart + g * SUPER
      pg = lax.rem(g, 2)
      png = 1 - pg
      @pl.when(g + 1 < nsuper)
      def _():
        nxt = rowbase + SUPER
        pltpu.async_copy(ei_hbm.at[0, pl.ds(nxt, SUPER)],
                         sidx_v.at[png], isem0.at[png])
        pltpu.async_copy(ei_hbm.at[1, pl.ds(nxt, SUPER)],
                         didx_v.at[png], isem1.at[png])

      pltpu.make_async_copy(ei_hbm.at[0, pl.ds(rowbase, SUPER)],
                            sidx_v.at[pg], isem0.at[pg]).wait()
      pltpu.make_async_copy(ei_hbm.at[1, pl.ds(rowbase, SUPER)],
                            didx_v.at[pg], isem1.at[pg]).wait()

      gd = [None] * SUPER
      sd = [None] * SUPER
      for j in range(3):
        gd[j] = pltpu.async_copy(feat_hbm.at[sidx_v.at[pg, j]], rows_v.at[j],
                                 gsem)
      for j in range(SUPER):
        b = j % 3
        gd[j].wait()
        sd[j] = pltpu.async_copy(rows_v.at[b], acc_sh.at[didx_v.at[pg, j]],
                                 ssem, add=True)
        for k in range(EW // 16):
          idx = didx_v[pg, j, pl.ds(k * 16, 16)]
          plsc.addupdate_scatter(cnt_v, [idx], ones16)
        if j + 3 < SUPER:
          sd[j].wait()
          gd[j + 3] = pltpu.async_copy(feat_hbm.at[sidx_v.at[pg, j + 3]],
                                       rows_v.at[b], gsem)
      for j in range(SUPER - 3, SUPER):
        sd[j].wait()
      return 0

    lax.fori_loop(0, nsuper, superchunk, 0)

  def writeback(p):
    pltpu.sync_copy(acc_sh.at[pl.ds(s * ROWS_T, ROWS_T)],
                    sums_hbm.at[p, c, pl.ds(s * ROWS_T, ROWS_T)])
    pltpu.sync_copy(cnt_v, cnts_hbm.at[p, c, s])

    @pl.when(s == NS - 1)
    def _():
      pltpu.sync_copy(acc_sh.at[pl.ds(NS * ROWS_T, REM)],
                      sums_hbm.at[p, c, pl.ds(NS * ROWS_T, REM)])

  zero_phase()
  plsc.subcore_barrier()
  ns0 = jnp.where(s == NS - 1, P0_SC + 1, P0_SC)
  rs0 = s * (P0_SC * SUPER)

  @pl.when(c == 0)
  def _():
    accum(e0_hbm, rs0, ns0)

  @pl.when(c == 1)
  def _():
    accum(e1_hbm, rs0, ns0)

  plsc.subcore_barrier()
  writeback(0)
  plsc.subcore_barrier()

  zero_phase()
  plsc.subcore_barrier()
  ns1 = jnp.where(wid < P1_XTRA, P1_SC + 1, P1_SC)
  rs1 = wid * (P1_SC * SUPER) + jnp.minimum(wid, P1_XTRA) * SUPER
  accum(e2_hbm, rs1, ns1)
  plsc.subcore_barrier()
  writeback(1)
  plsc.subcore_barrier()


_sc_call = pl.kernel(
    _sc_body,
    out_type=[
        jax.ShapeDtypeStruct((2, NC, N, D), jnp.float32),
        jax.ShapeDtypeStruct((2, NC, NS, NACC), jnp.float32),
    ],
    mesh=plsc.VectorSubcoreMesh(core_axis_name="c", subcore_axis_name="s"),
    compiler_params=pltpu.CompilerParams(needs_layout_passes=False),
    scratch_types=[
        pltpu.VMEM_SHARED((NACC, D), jnp.float32),
        pltpu.VMEM((3, EW, D), jnp.float32),
        pltpu.VMEM((2, SUPER, EW), jnp.int32),
        pltpu.VMEM((2, SUPER, EW), jnp.int32),
        pltpu.VMEM((NACC,), jnp.float32),
        pltpu.SemaphoreType.DMA,
        pltpu.SemaphoreType.DMA,
        pltpu.SemaphoreType.DMA((2,)),
        pltpu.SemaphoreType.DMA((2,)),
    ],
)


BLK = 1000


def _tc_body(sums_ref, cnt_ref, w_ref, b_ref, out_ref):
  acc = None
  for e in range(3):
    if e < 2:
      ssum = sums_ref[e]
      cnt = jnp.sum(cnt_ref[e], axis=1, keepdims=True)
    else:
      ssum = sums_ref[2] + sums_ref[3]
      cnt = (jnp.sum(cnt_ref[2], axis=1, keepdims=True)
             + jnp.sum(cnt_ref[3], axis=1, keepdims=True))
    scale = 1.0 / jnp.maximum(cnt, 1.0)
    mask = (cnt > 0.5).astype(jnp.float32)
    term = jnp.dot(ssum * scale, w_ref[e], preferred_element_type=jnp.float32)
    term = term + mask * b_ref[e]
    acc = term if acc is None else acc + term
  out_ref[...] = jnp.maximum(acc, 0.0)


def _tc_call(sums, cnts, ws, bs):
  return pl.pallas_call(
      _tc_body,
      grid=(N // BLK,),
      in_specs=[
          pl.BlockSpec((4, BLK, D), lambda i: (0, i, 0)),
          pl.BlockSpec((4, BLK, NS), lambda i: (0, i, 0)),
          pl.BlockSpec((3, D, D), lambda i: (0, 0, 0)),
          pl.BlockSpec((3, 1, D), lambda i: (0, 0, 0)),
      ],
      out_specs=pl.BlockSpec((BLK, D), lambda i: (i, 0)),
      out_shape=jax.ShapeDtypeStruct((N, D), jnp.float32),
  )(sums, cnts, ws, bs)


@jax.jit
def kernel(features, edge_index_e0, edge_index_e1, edge_index_e2,
           W_e0, b_e0, W_e1, b_e1, W_e2, b_e2):
  e0r = edge_index_e0.reshape(2, ROWS_ALL, EW)
  e1r = edge_index_e1.reshape(2, ROWS_ALL, EW)
  e2r = edge_index_e2.reshape(2, ROWS_ALL, EW)
  zz_h = jnp.zeros((ZROWS, D), jnp.float32)
  zc_h = jnp.zeros((NACC,), jnp.float32)

  sums, cnts = _sc_call(features, e0r, e1r, e2r, zz_h, zc_h)

  cnt_t = jnp.transpose(cnts.reshape(2 * NC, NS, NACC), (0, 2, 1))
  ws = jnp.stack([W_e0, W_e1, W_e2])
  bs = jnp.stack([b_e0, b_e1, b_e2]).reshape(3, 1, D)
  return _tc_call(sums.reshape(2 * NC, N, D), cnt_t, ws, bs)
